# baseline (speedup 1.0000x reference)
import functools

import jax
import jax.numpy as jnp
from jax import lax
from jax.experimental import pallas as pl
from jax.experimental.pallas import tpu as pltpu
from jax.experimental.pallas import tpu_sc as plsc

N_NODES = 50000
N_EDGES = 800000
HEADS = 4
HID = 32

NC = 2
NS = 16
NW = NC * NS
L = 16

N_PAD = 50176
E_TOT = N_EDGES + N_NODES
E_PAD = 860160
EB = E_PAD // NW
B_E = 128
NBATCH = EB // B_E
HALF = N_PAD // 2
ACC_ROWS = HALF + 16
RPT = HALF // NS


def _mesh():
    return plsc.VectorSubcoreMesh(core_axis_name="c", subcore_axis_name="s",
                                  num_cores=NC, num_subcores=NS)


def _sc_params():
    return pltpu.CompilerParams(needs_layout_passes=False,
                                use_tc_tiling_on_sc=False)


def _edge_b_body(H, xl_ref, xr_ref, src_ref, dst_ref, att_ref,
                 ex_ref, dpart_ref,
                 att_v, srcv, dstv, adjv, xlv, xrv, tv, exv, denv,
                 sx0, sr0, sx1, sr1, si0, si1):
    semx = (sx0, sx1)
    semr = (sr0, sr1)
    semi = (si0, si1)
    wid = lax.axis_index("s") * NC + lax.axis_index("c")
    ebase = wid * EB
    pltpu.sync_copy(att_ref, att_v)

    for h in range(H):
        def zero_body(i, c):
            denv[pl.ds(i * L, L)] = jnp.zeros((L,), jnp.float32)
            return c
        lax.fori_loop(0, N_PAD // L, zero_body, 0)

        att0 = att_v[pl.ds(h * 32, L)]
        att1 = att_v[pl.ds(h * 32 + L, L)]

        def issue(j, b):
            base = ebase + j * B_E
            ci = pltpu.async_copy(src_ref.at[pl.ds(base, B_E)], srcv.at[b],
                                  semi[b])
            cj = pltpu.async_copy(dst_ref.at[pl.ds(base, B_E)], dstv.at[b],
                                  semi[b])
            ci.wait()
            cj.wait()

            def adj_body(g, c2):
                sl = pl.ds(g * L, L)
                adjv[2 * b, sl] = srcv[b, sl] + h * N_PAD
                adjv[2 * b + 1, sl] = dstv[b, sl] + h * N_PAD
                return c2
            lax.fori_loop(0, B_E // L, adj_body, 0)
            pltpu.async_copy(xl_ref.at[adjv.at[2 * b]], xlv.at[b], semx[b])
            pltpu.async_copy(xr_ref.at[adjv.at[2 * b + 1]], xrv.at[b],
                             semr[b])

        def crunch(j, b):
            pltpu.make_async_copy(
                xl_ref.at[adjv.at[2 * b]], xlv.at[b], semx[b]).wait()
            pltpu.make_async_copy(
                xr_ref.at[adjv.at[2 * b + 1]], xrv.at[b], semr[b]).wait()

            def t_body(r4, c2):
                for u in range(4):
                    r = r4 * 4 + u
                    a = xlv[b, r, pl.ds(0, L)] + xrv[b, r, pl.ds(0, L)]
                    a = jnp.where(a >= 0.0, a, a * jnp.float32(0.2))
                    tv[pl.ds(r * 32, L)] = a * att0
                    bb = xlv[b, r, pl.ds(L, L)] + xrv[b, r, pl.ds(L, L)]
                    bb = jnp.where(bb >= 0.0, bb, bb * jnp.float32(0.2))
                    tv[pl.ds(r * 32 + L, L)] = bb * att1
                return c2
            lax.fori_loop(0, B_E // 4, t_body, 0)

            rowoff = lax.iota(jnp.int32, L) * 32

            def lg_body(g, c2):
                bidx = rowoff + g * (L * 32)
                acc = jnp.zeros((L,), jnp.float32)
                for col in range(32):
                    acc = acc + plsc.load_gather(tv, [bidx + col])
                e16 = jnp.exp(acc)
                exv[pl.ds(g * L, L)] = e16
                d16 = dstv[b, pl.ds(g * L, L)]
                plsc.addupdate_scatter(denv, [d16], e16)
                return c2
            lax.fori_loop(0, B_E // L, lg_body, 0)

            base = ebase + j * B_E
            pltpu.sync_copy(exv, ex_ref.at[pl.ds(h * E_PAD + base, B_E)])

        issue(0, 0)

        def pipe_body(jj, c):
            j0 = 2 * jj
            issue(j0 + 1, 1)
            crunch(j0, 0)

            @pl.when(jj < NBATCH // 2 - 1)
            def _():
                issue(j0 + 2, 0)
            crunch(j0 + 1, 1)
            return c
        lax.fori_loop(0, NBATCH // 2, pipe_body, 0)

        pltpu.sync_copy(denv,
                        dpart_ref.at[pl.ds((h * NW + wid) * N_PAD, N_PAD)])


def _edge_b(xl_flat, xr_flat, src, dst, att, H):
    body = functools.partial(_edge_b_body, H)
    f = pl.kernel(
        body,
        out_type=(
            jax.ShapeDtypeStruct((H * E_PAD,), jnp.float32),
            jax.ShapeDtypeStruct((H * NW * N_PAD,), jnp.float32),
        ),
        mesh=_mesh(),
        compiler_params=_sc_params(),
        scratch_types=[
            pltpu.VMEM((H * 32,), jnp.float32),
            pltpu.VMEM((2, B_E), jnp.int32),
            pltpu.VMEM((2, B_E), jnp.int32),
            pltpu.VMEM((4, B_E), jnp.int32),
            pltpu.VMEM((2, B_E, 32), jnp.float32),
            pltpu.VMEM((2, B_E, 32), jnp.float32),
            pltpu.VMEM((B_E * 32,), jnp.float32),
            pltpu.VMEM((B_E,), jnp.float32),
            pltpu.VMEM((N_PAD,), jnp.float32),
            pltpu.SemaphoreType.DMA,
            pltpu.SemaphoreType.DMA,
            pltpu.SemaphoreType.DMA,
            pltpu.SemaphoreType.DMA,
            pltpu.SemaphoreType.DMA,
            pltpu.SemaphoreType.DMA,
        ],
    )
    return f(xl_flat, xr_flat, src, dst, att.reshape(-1))


def _edge_d_body(H, xl_ref, src_ref, dst_ref, ex_ref, den_ref,
                 outp_ref,
                 srcv, adjv, dstv, drel, exv, wv, rowsv, denv, zv,
                 acc_sh, sg0, sg1, ss0, ss1, si0, si1):
    semg = (sg0, sg1)
    sems = (ss0, ss1)
    semi = (si0, si1)
    cid = lax.axis_index("c")
    sid = lax.axis_index("s")
    ebase = sid * (E_PAD // NS)
    rbase = cid * HALF

    def zz_body(r, c):
        zv[r, pl.ds(0, L)] = jnp.zeros((L,), jnp.float32)
        zv[r, pl.ds(L, L)] = jnp.zeros((L,), jnp.float32)
        return c
    lax.fori_loop(0, RPT // 8, zz_body, 0)

    for h in range(H):
        for q in range(8):
            pltpu.sync_copy(
                zv, acc_sh.at[pl.ds(sid * RPT + q * (RPT // 8), RPT // 8)])

        @pl.when(sid == 0)
        def _():
            pltpu.sync_copy(zv.at[pl.ds(0, 16)],
                            acc_sh.at[pl.ds(HALF, 16)])
        pltpu.sync_copy(den_ref.at[pl.ds(h * N_PAD, N_PAD)], denv)
        plsc.subcore_barrier()

        nb_d = E_PAD // NS // B_E

        def issue(j, b, first):
            base = ebase + j * B_E
            ci = pltpu.async_copy(src_ref.at[pl.ds(base, B_E)], srcv.at[b],
                                  semi[b])
            cj = pltpu.async_copy(dst_ref.at[pl.ds(base, B_E)], dstv.at[b],
                                  semi[b])
            ck = pltpu.async_copy(ex_ref.at[pl.ds(h * E_PAD + base, B_E)],
                                  exv.at[b], semi[b])
            ci.wait()
            cj.wait()
            ck.wait()
            if not first:
                pltpu.make_async_copy(
                    rowsv.at[b], acc_sh.at[drel.at[b]], sems[b]).wait()

            def adj_body(g, c2):
                sl = pl.ds(g * L, L)
                adjv[b, sl] = srcv[b, sl] + h * N_PAD
                rel = dstv[b, sl] - rbase
                oob = (rel < 0) | (rel >= HALF)
                drel[b, sl] = jnp.where(oob, HALF, rel)
                return c2
            lax.fori_loop(0, B_E // L, adj_body, 0)
            pltpu.async_copy(xl_ref.at[adjv.at[b]], rowsv.at[b], semg[b])

        def crunch(j, b):
            pltpu.make_async_copy(
                xl_ref.at[adjv.at[b]], rowsv.at[b], semg[b]).wait()

            def w_body(g, c2):
                d16 = plsc.load_gather(denv, [dstv[b, pl.ds(g * L, L)]])
                wv[pl.ds(g * L, L)] = exv[b, pl.ds(g * L, L)] / (
                    d16 + jnp.float32(1e-16))
                return c2
            lax.fori_loop(0, B_E // L, w_body, 0)

            def scale_body(r4, c2):
                for u in range(4):
                    r = r4 * 4 + u
                    ws = plsc.load_gather(wv, [jnp.full((L,), r, jnp.int32)])
                    rowsv[b, r, pl.ds(0, L)] = rowsv[b, r, pl.ds(0, L)] * ws
                    rowsv[b, r, pl.ds(L, L)] = rowsv[b, r, pl.ds(L, L)] * ws
                return c2
            lax.fori_loop(0, B_E // 4, scale_body, 0)

            pltpu.async_copy(rowsv.at[b], acc_sh.at[drel.at[b]], sems[b],
                             add=True)

        issue(0, 0, True)

        def pipe_first(_, c):
            issue(1, 1, True)
            crunch(0, 0)
            issue(2, 0, False)
            crunch(1, 1)
            return c
        lax.fori_loop(0, 1, pipe_first, 0)

        def pipe_body(jj, c):
            j0 = 2 * jj
            issue(j0 + 1, 1, False)
            crunch(j0, 0)

            @pl.when(jj < nb_d // 2 - 1)
            def _():
                issue(j0 + 2, 0, False)
            crunch(j0 + 1, 1)
            return c
        lax.fori_loop(1, nb_d // 2, pipe_body, 0)

        for b in range(2):
            pltpu.make_async_copy(
                rowsv.at[b], acc_sh.at[drel.at[b]], sems[b]).wait()
        plsc.subcore_barrier()

        rs = sid * RPT
        pltpu.sync_copy(
            acc_sh.at[pl.ds(rs, RPT)],
            outp_ref.at[pl.ds(h * N_PAD + rbase + rs, RPT)])
        plsc.subcore_barrier()


def _edge_d(xl_flat, src, dst, ex, den, H):
    body = functools.partial(_edge_d_body, H)
    f = pl.kernel(
        body,
        out_type=jax.ShapeDtypeStruct((H * N_PAD, 32), jnp.float32),
        mesh=_mesh(),
        compiler_params=_sc_params(),
        scratch_types=[
            pltpu.VMEM((3, B_E), jnp.int32),
            pltpu.VMEM((3, B_E), jnp.int32),
            pltpu.VMEM((3, B_E), jnp.int32),
            pltpu.VMEM((3, B_E), jnp.int32),
            pltpu.VMEM((3, B_E), jnp.float32),
            pltpu.VMEM((B_E,), jnp.float32),
            pltpu.VMEM((3, B_E, 32), jnp.float32),
            pltpu.VMEM((N_PAD,), jnp.float32),
            pltpu.VMEM((RPT // 8, 32), jnp.float32),
            pltpu.VMEM_SHARED((ACC_ROWS, 32), jnp.float32),
        ] + [pltpu.SemaphoreType.DMA] * 6,
    )
    return f(xl_flat, src, dst, ex, den)


def _embed_body(x_ref, wemb_ref, bemb_ref, wl_ref, wr_ref, oxl_ref, oxr_ref):
    h0 = jax.nn.relu(
        jnp.dot(x_ref[...], wemb_ref[...],
                preferred_element_type=jnp.float32) + bemb_ref[...])
    xl = jnp.dot(h0, wl_ref[...], preferred_element_type=jnp.float32)
    xr = jnp.dot(h0, wr_ref[...], preferred_element_type=jnp.float32)
    for h in range(HEADS):
        oxl_ref[h, :, :] = xl[:, h * 32:(h + 1) * 32]
        oxr_ref[h, :, :] = xr[:, h * 32:(h + 1) * 32]


def _embed(x2p, Wemb, bemb, W1l, W1r):
    nb = 512
    grid = (N_PAD // nb,)
    return pl.pallas_call(
        _embed_body,
        grid=grid,
        in_specs=[
            pl.BlockSpec((nb, 10), lambda i: (i, 0)),
            pl.BlockSpec((10, 32), lambda i: (0, 0)),
            pl.BlockSpec((1, 32), lambda i: (0, 0)),
            pl.BlockSpec((32, 128), lambda i: (0, 0)),
            pl.BlockSpec((32, 128), lambda i: (0, 0)),
        ],
        out_specs=[
            pl.BlockSpec((HEADS, nb, 32), lambda i: (0, i, 0)),
            pl.BlockSpec((HEADS, nb, 32), lambda i: (0, i, 0)),
        ],
        out_shape=[
            jax.ShapeDtypeStruct((HEADS, N_PAD, 32), jnp.float32),
            jax.ShapeDtypeStruct((HEADS, N_PAD, 32), jnp.float32),
        ],
    )(x2p, Wemb, bemb.reshape(1, 32), W1l, W1r)


def _denom_combine_body(dp_ref, o_ref):
    o_ref[...] = jnp.sum(dp_ref[...], axis=1)


def _denom_combine(dpart, H):
    nb = 1024
    return pl.pallas_call(
        _denom_combine_body,
        grid=(N_PAD // nb,),
        in_specs=[pl.BlockSpec((H, NW, nb), lambda i: (0, 0, i))],
        out_specs=pl.BlockSpec((H, nb), lambda i: (0, i)),
        out_shape=jax.ShapeDtypeStruct((H, N_PAD), jnp.float32),
    )(dpart.reshape(H, NW, N_PAD)).reshape(-1)


def _combine_e_body(Hn, p_ref, b_ref, wl_ref, wr_ref, oxl_ref, oxr_ref):
    s = p_ref[...]
    hb = jnp.concatenate([s[h] for h in range(HEADS)], axis=1) + b_ref[...]
    hb = jnp.where(hb > 0.0, hb, jnp.exp(jnp.minimum(hb, 0.0)) - 1.0)
    xl = jnp.dot(hb, wl_ref[...], preferred_element_type=jnp.float32)
    xr = jnp.dot(hb, wr_ref[...], preferred_element_type=jnp.float32)
    for h in range(Hn):
        oxl_ref[h, :, :] = xl[:, h * 32:(h + 1) * 32]
        oxr_ref[h, :, :] = xr[:, h * 32:(h + 1) * 32]


def _combine_e(outp, bias, Wl, Wr, Hn):
    nb = 512
    body = functools.partial(_combine_e_body, Hn)
    return pl.pallas_call(
        body,
        grid=(N_PAD // nb,),
        in_specs=[
            pl.BlockSpec((HEADS, nb, 32), lambda i: (0, i, 0)),
            pl.BlockSpec((1, 128), lambda i: (0, 0)),
            pl.BlockSpec((128, 32 * Hn), lambda i: (0, 0)),
            pl.BlockSpec((128, 32 * Hn), lambda i: (0, 0)),
        ],
        out_specs=[
            pl.BlockSpec((Hn, nb, 32), lambda i: (0, i, 0)),
            pl.BlockSpec((Hn, nb, 32), lambda i: (0, i, 0)),
        ],
        out_shape=[
            jax.ShapeDtypeStruct((Hn, N_PAD, 32), jnp.float32),
            jax.ShapeDtypeStruct((Hn, N_PAD, 32), jnp.float32),
        ],
    )(outp.reshape(HEADS, N_PAD, 32), bias.reshape(1, 128), Wl, Wr)


def _focal_body(op3_ref, focal_ref, o_ref, fv, r0, sem):
    wid = lax.axis_index("s") * NC + lax.axis_index("c")

    @pl.when(wid == 0)
    def _():
        pltpu.sync_copy(focal_ref, fv)
        pltpu.async_copy(op3_ref.at[fv], r0, sem).wait()
        pltpu.sync_copy(r0, o_ref)


def _focal_gather(outp3, focal_idx):
    f = pl.kernel(
        _focal_body,
        out_type=jax.ShapeDtypeStruct((64, 32), jnp.float32),
        mesh=_mesh(),
        compiler_params=_sc_params(),
        scratch_types=[
            pltpu.VMEM((64,), jnp.int32),
            pltpu.VMEM((64, 32), jnp.float32),
            pltpu.SemaphoreType.DMA,
        ],
    )
    return f(outp3, focal_idx.astype(jnp.int32))


def _fc_body(rows_ref, b3_ref, w_ref, bfc_ref, o_ref):
    comb = rows_ref[...] + b3_ref[...]
    o_ref[...] = jnp.dot(comb, w_ref[...],
                         preferred_element_type=jnp.float32) + bfc_ref[...]


def _fc(focal_idx, outp3, b3, Wfc_top, bfc):
    rows = _focal_gather(outp3, focal_idx)
    return pl.pallas_call(
        _fc_body,
        out_shape=jax.ShapeDtypeStruct((64, 60), jnp.float32),
    )(rows, b3.reshape(1, 32), Wfc_top, bfc.reshape(1, 60))


def _gat_layer(xl, xr, src, dst, att, H):
    xl_flat = xl.reshape(H * N_PAD, 32)
    xr_flat = xr.reshape(H * N_PAD, 32)
    ex, dpart = _edge_b(xl_flat, xr_flat, src, dst, att, H)
    den = _denom_combine(dpart, H)
    return _edge_d(xl_flat, src, dst, ex, den, H)


def kernel(x, edge_index, focal_idx, Wemb, bemb, W1l, W1r, att1, b1,
           W2l, W2r, att2, b2, W3l, W3r, att3, b3, Wfc, bfc):
    N = x.shape[0]
    loops = jnp.arange(N, dtype=jnp.int32)
    src = jnp.concatenate([edge_index[0].astype(jnp.int32), loops])
    dst = jnp.concatenate([edge_index[1].astype(jnp.int32), loops])
    pad = E_PAD - E_TOT
    src = jnp.concatenate([src, jnp.full((pad,), N, jnp.int32)])
    dst = jnp.concatenate([dst, jnp.full((pad,), N, jnp.int32)])

    x2 = x.reshape(N, 10)
    x2p = jnp.zeros((N_PAD, 10), jnp.float32).at[:N].set(x2)

    xl1, xr1 = _embed(x2p, Wemb, bemb, W1l, W1r)
    op1 = _gat_layer(xl1, xr1, src, dst, att1, HEADS)
    xl2, xr2 = _combine_e(op1, b1, W2l, W2r, HEADS)
    op2 = _gat_layer(xl2, xr2, src, dst, att2, HEADS)
    xl3, xr3 = _combine_e(op2, b2, W3l, W3r, 1)
    op3 = _gat_layer(xl3, xr3, src, dst, att3, 1)
    return _fc(focal_idx, op3, b3, Wfc[:HID], bfc)


# speedup vs baseline: 1.0756x; 1.0756x over previous
import functools

import jax
import jax.numpy as jnp
from jax import lax
from jax.experimental import pallas as pl
from jax.experimental.pallas import tpu as pltpu
from jax.experimental.pallas import tpu_sc as plsc

N_NODES = 50000
N_EDGES = 800000
HEADS = 4
HID = 32

NC = 2
NS = 16
NW = NC * NS
L = 16

N_PAD = 50176
E_TOT = N_EDGES + N_NODES
E_PAD = 851968
EB = E_PAD // NW
B_E = 128
NBATCH = EB // B_E
HALF = N_PAD // 2
ACC_ROWS = HALF + 16
RPT = HALF // NS


def _mesh():
    return plsc.VectorSubcoreMesh(core_axis_name="c", subcore_axis_name="s",
                                  num_cores=NC, num_subcores=NS)


def _sc_params():
    return pltpu.CompilerParams(needs_layout_passes=False,
                                use_tc_tiling_on_sc=False)


def _edge_b_body(H, xl_ref, xr_ref, src_ref, dst_ref, att_ref,
                 ex_ref, dpart_ref,
                 att_v, srcv, dstv, adjv, xlv, xrv, tv, exv, denv,
                 sx0, sr0, sx1, sr1, si0, si1):
    semx = (sx0, sx1)
    semr = (sr0, sr1)
    semi = (si0, si1)
    wid = lax.axis_index("s") * NC + lax.axis_index("c")
    ebase = wid * EB
    pltpu.sync_copy(att_ref, att_v)

    for h in range(H):
        def zero_body(i, c):
            denv[pl.ds(i * L, L)] = jnp.zeros((L,), jnp.float32)
            return c
        lax.fori_loop(0, N_PAD // L, zero_body, 0)

        att0 = att_v[pl.ds(h * 32, L)]
        att1 = att_v[pl.ds(h * 32 + L, L)]

        def issue(j, b):
            base = ebase + j * B_E
            ci = pltpu.async_copy(src_ref.at[pl.ds(base, B_E)], srcv.at[b],
                                  semi[b])
            cj = pltpu.async_copy(dst_ref.at[pl.ds(base, B_E)], dstv.at[b],
                                  semi[b])
            ci.wait()
            cj.wait()

            def adj_body(g, c2):
                sl = pl.ds(g * L, L)
                adjv[2 * b, sl] = srcv[b, sl] + h * N_PAD
                adjv[2 * b + 1, sl] = dstv[b, sl] + h * N_PAD
                return c2
            lax.fori_loop(0, B_E // L, adj_body, 0)
            pltpu.async_copy(xl_ref.at[adjv.at[2 * b]], xlv.at[b], semx[b])
            pltpu.async_copy(xr_ref.at[adjv.at[2 * b + 1]], xrv.at[b],
                             semr[b])

        def crunch(j, b):
            pltpu.make_async_copy(
                xl_ref.at[adjv.at[2 * b]], xlv.at[b], semx[b]).wait()
            pltpu.make_async_copy(
                xr_ref.at[adjv.at[2 * b + 1]], xrv.at[b], semr[b]).wait()

            def t_body(r4, c2):
                for u in range(4):
                    r = r4 * 4 + u
                    a = xlv[b, r, pl.ds(0, L)] + xrv[b, r, pl.ds(0, L)]
                    a = jnp.where(a >= 0.0, a, a * jnp.float32(0.2))
                    tv[pl.ds(r * 32, L)] = a * att0
                    bb = xlv[b, r, pl.ds(L, L)] + xrv[b, r, pl.ds(L, L)]
                    bb = jnp.where(bb >= 0.0, bb, bb * jnp.float32(0.2))
                    tv[pl.ds(r * 32 + L, L)] = bb * att1
                return c2
            lax.fori_loop(0, B_E // 4, t_body, 0)

            rowoff = lax.iota(jnp.int32, L) * 32

            def lg_body(g, c2):
                bidx = rowoff + g * (L * 32)
                acc = jnp.zeros((L,), jnp.float32)
                for col in range(32):
                    acc = acc + plsc.load_gather(tv, [bidx + col])
                e16 = jnp.exp(acc)
                exv[pl.ds(g * L, L)] = e16
                d16 = dstv[b, pl.ds(g * L, L)]
                plsc.addupdate_scatter(denv, [d16], e16)
                return c2
            lax.fori_loop(0, B_E // L, lg_body, 0)

            base = ebase + j * B_E
            pltpu.sync_copy(exv, ex_ref.at[pl.ds(h * E_PAD + base, B_E)])

        issue(0, 0)

        def pipe_body(jj, c):
            j0 = 2 * jj
            issue(j0 + 1, 1)
            crunch(j0, 0)

            @pl.when(jj < NBATCH // 2 - 1)
            def _():
                issue(j0 + 2, 0)
            crunch(j0 + 1, 1)
            return c
        lax.fori_loop(0, NBATCH // 2, pipe_body, 0)

        pltpu.sync_copy(denv,
                        dpart_ref.at[pl.ds((h * NW + wid) * N_PAD, N_PAD)])


def _edge_b(xl_flat, xr_flat, src, dst, att, H):
    body = functools.partial(_edge_b_body, H)
    f = pl.kernel(
        body,
        out_type=(
            jax.ShapeDtypeStruct((H * E_PAD,), jnp.float32),
            jax.ShapeDtypeStruct((H * NW * N_PAD,), jnp.float32),
        ),
        mesh=_mesh(),
        compiler_params=_sc_params(),
        scratch_types=[
            pltpu.VMEM((H * 32,), jnp.float32),
            pltpu.VMEM((2, B_E), jnp.int32),
            pltpu.VMEM((2, B_E), jnp.int32),
            pltpu.VMEM((4, B_E), jnp.int32),
            pltpu.VMEM((2, B_E, 32), jnp.float32),
            pltpu.VMEM((2, B_E, 32), jnp.float32),
            pltpu.VMEM((B_E * 32,), jnp.float32),
            pltpu.VMEM((B_E,), jnp.float32),
            pltpu.VMEM((N_PAD,), jnp.float32),
            pltpu.SemaphoreType.DMA,
            pltpu.SemaphoreType.DMA,
            pltpu.SemaphoreType.DMA,
            pltpu.SemaphoreType.DMA,
            pltpu.SemaphoreType.DMA,
            pltpu.SemaphoreType.DMA,
        ],
    )
    return f(xl_flat, xr_flat, src, dst, att.reshape(-1))


def _edge_d_body(H, xl_ref, src_ref, dst_ref, ex_ref, den_ref,
                 outp_ref,
                 srcv, adjv, dstv, drel, exv, wv, rowsv, denv, zv,
                 acc_sh, sg0, sg1, ss0, ss1, si0, si1):
    semg = (sg0, sg1)
    sems = (ss0, ss1)
    semi = (si0, si1)
    cid = lax.axis_index("c")
    sid = lax.axis_index("s")
    ebase = sid * (E_PAD // NS)
    rbase = cid * HALF

    def zz_body(r, c):
        zv[r, pl.ds(0, L)] = jnp.zeros((L,), jnp.float32)
        zv[r, pl.ds(L, L)] = jnp.zeros((L,), jnp.float32)
        return c
    lax.fori_loop(0, RPT // 4, zz_body, 0)

    for h in range(H):
        for q in range(4):
            pltpu.sync_copy(
                zv, acc_sh.at[pl.ds(sid * RPT + q * (RPT // 4), RPT // 4)])

        @pl.when(sid == 0)
        def _():
            pltpu.sync_copy(zv.at[pl.ds(0, 16)],
                            acc_sh.at[pl.ds(HALF, 16)])
        pltpu.sync_copy(den_ref.at[pl.ds(h * N_PAD, N_PAD)], denv)
        plsc.subcore_barrier()

        nb_d = E_PAD // NS // B_E

        def issue(j, b, first):
            base = ebase + j * B_E
            ci = pltpu.async_copy(src_ref.at[pl.ds(base, B_E)], srcv.at[b],
                                  semi[b])
            cj = pltpu.async_copy(dst_ref.at[pl.ds(base, B_E)], dstv.at[b],
                                  semi[b])
            ck = pltpu.async_copy(ex_ref.at[pl.ds(h * E_PAD + base, B_E)],
                                  exv.at[b], semi[b])
            ci.wait()
            cj.wait()
            ck.wait()
            if not first:
                pltpu.make_async_copy(
                    rowsv.at[b], acc_sh.at[drel.at[b]], sems[b]).wait()

            def adj_body(g, c2):
                sl = pl.ds(g * L, L)
                adjv[b, sl] = srcv[b, sl] + h * N_PAD
                rel = dstv[b, sl] - rbase
                oob = (rel < 0) | (rel >= HALF)
                drel[b, sl] = jnp.where(oob, HALF, rel)
                return c2
            lax.fori_loop(0, B_E // L, adj_body, 0)
            pltpu.async_copy(xl_ref.at[adjv.at[b]], rowsv.at[b], semg[b])

        def crunch(j, b):
            pltpu.make_async_copy(
                xl_ref.at[adjv.at[b]], rowsv.at[b], semg[b]).wait()

            def w_body(g, c2):
                d16 = plsc.load_gather(denv, [dstv[b, pl.ds(g * L, L)]])
                wv[pl.ds(g * L, L)] = exv[b, pl.ds(g * L, L)] / (
                    d16 + jnp.float32(1e-16))
                return c2
            lax.fori_loop(0, B_E // L, w_body, 0)

            def scale_body(r4, c2):
                for u in range(4):
                    r = r4 * 4 + u
                    ws = plsc.load_gather(wv, [jnp.full((L,), r, jnp.int32)])
                    rowsv[b, r, pl.ds(0, L)] = rowsv[b, r, pl.ds(0, L)] * ws
                    rowsv[b, r, pl.ds(L, L)] = rowsv[b, r, pl.ds(L, L)] * ws
                return c2
            lax.fori_loop(0, B_E // 4, scale_body, 0)

            pltpu.async_copy(rowsv.at[b], acc_sh.at[drel.at[b]], sems[b],
                             add=True)

        issue(0, 0, True)

        def pipe_first(_, c):
            issue(1, 1, True)
            crunch(0, 0)
            issue(2, 0, False)
            crunch(1, 1)
            return c
        lax.fori_loop(0, 1, pipe_first, 0)

        def pipe_body(jj, c):
            j0 = 2 * jj
            issue(j0 + 1, 1, False)
            crunch(j0, 0)

            @pl.when(jj < nb_d // 2 - 1)
            def _():
                issue(j0 + 2, 0, False)
            crunch(j0 + 1, 1)
            return c
        lax.fori_loop(1, nb_d // 2, pipe_body, 0)

        for b in range(2):
            pltpu.make_async_copy(
                rowsv.at[b], acc_sh.at[drel.at[b]], sems[b]).wait()
        plsc.subcore_barrier()

        rs = sid * RPT
        pltpu.sync_copy(
            acc_sh.at[pl.ds(rs, RPT)],
            outp_ref.at[pl.ds(h * N_PAD + rbase + rs, RPT)])
        plsc.subcore_barrier()


def _edge_d(xl_flat, src, dst, ex, den, H):
    body = functools.partial(_edge_d_body, H)
    f = pl.kernel(
        body,
        out_type=jax.ShapeDtypeStruct((H * N_PAD, 32), jnp.float32),
        mesh=_mesh(),
        compiler_params=_sc_params(),
        scratch_types=[
            pltpu.VMEM((3, B_E), jnp.int32),
            pltpu.VMEM((3, B_E), jnp.int32),
            pltpu.VMEM((3, B_E), jnp.int32),
            pltpu.VMEM((3, B_E), jnp.int32),
            pltpu.VMEM((3, B_E), jnp.float32),
            pltpu.VMEM((B_E,), jnp.float32),
            pltpu.VMEM((3, B_E, 32), jnp.float32),
            pltpu.VMEM((N_PAD,), jnp.float32),
            pltpu.VMEM((RPT // 4, 32), jnp.float32),
            pltpu.VMEM_SHARED((ACC_ROWS, 32), jnp.float32),
        ] + [pltpu.SemaphoreType.DMA] * 6,
    )
    return f(xl_flat, src, dst, ex, den)


def _embed_body(x_ref, wemb_ref, bemb_ref, wl_ref, wr_ref, oxl_ref, oxr_ref):
    h0 = jax.nn.relu(
        jnp.dot(x_ref[...], wemb_ref[...],
                preferred_element_type=jnp.float32) + bemb_ref[...])
    xl = jnp.dot(h0, wl_ref[...], preferred_element_type=jnp.float32)
    xr = jnp.dot(h0, wr_ref[...], preferred_element_type=jnp.float32)
    for h in range(HEADS):
        oxl_ref[h, :, :] = xl[:, h * 32:(h + 1) * 32]
        oxr_ref[h, :, :] = xr[:, h * 32:(h + 1) * 32]


def _embed(x2p, Wemb, bemb, W1l, W1r):
    nb = 512
    grid = (N_PAD // nb,)
    return pl.pallas_call(
        _embed_body,
        grid=grid,
        in_specs=[
            pl.BlockSpec((nb, 10), lambda i: (i, 0)),
            pl.BlockSpec((10, 32), lambda i: (0, 0)),
            pl.BlockSpec((1, 32), lambda i: (0, 0)),
            pl.BlockSpec((32, 128), lambda i: (0, 0)),
            pl.BlockSpec((32, 128), lambda i: (0, 0)),
        ],
        out_specs=[
            pl.BlockSpec((HEADS, nb, 32), lambda i: (0, i, 0)),
            pl.BlockSpec((HEADS, nb, 32), lambda i: (0, i, 0)),
        ],
        out_shape=[
            jax.ShapeDtypeStruct((HEADS, N_PAD, 32), jnp.float32),
            jax.ShapeDtypeStruct((HEADS, N_PAD, 32), jnp.float32),
        ],
    )(x2p, Wemb, bemb.reshape(1, 32), W1l, W1r)


def _denom_combine_body(dp_ref, o_ref):
    o_ref[...] = jnp.sum(dp_ref[...], axis=1)


def _denom_combine(dpart, H):
    nb = 1024
    return pl.pallas_call(
        _denom_combine_body,
        grid=(N_PAD // nb,),
        in_specs=[pl.BlockSpec((H, NW, nb), lambda i: (0, 0, i))],
        out_specs=pl.BlockSpec((H, nb), lambda i: (0, i)),
        out_shape=jax.ShapeDtypeStruct((H, N_PAD), jnp.float32),
    )(dpart.reshape(H, NW, N_PAD)).reshape(-1)


def _combine_e_body(Hn, p_ref, b_ref, wl_ref, wr_ref, oxl_ref, oxr_ref):
    s = p_ref[...]
    hb = jnp.concatenate([s[h] for h in range(HEADS)], axis=1) + b_ref[...]
    hb = jnp.where(hb > 0.0, hb, jnp.exp(jnp.minimum(hb, 0.0)) - 1.0)
    xl = jnp.dot(hb, wl_ref[...], preferred_element_type=jnp.float32)
    xr = jnp.dot(hb, wr_ref[...], preferred_element_type=jnp.float32)
    for h in range(Hn):
        oxl_ref[h, :, :] = xl[:, h * 32:(h + 1) * 32]
        oxr_ref[h, :, :] = xr[:, h * 32:(h + 1) * 32]


def _combine_e(outp, bias, Wl, Wr, Hn):
    nb = 512
    body = functools.partial(_combine_e_body, Hn)
    return pl.pallas_call(
        body,
        grid=(N_PAD // nb,),
        in_specs=[
            pl.BlockSpec((HEADS, nb, 32), lambda i: (0, i, 0)),
            pl.BlockSpec((1, 128), lambda i: (0, 0)),
            pl.BlockSpec((128, 32 * Hn), lambda i: (0, 0)),
            pl.BlockSpec((128, 32 * Hn), lambda i: (0, 0)),
        ],
        out_specs=[
            pl.BlockSpec((Hn, nb, 32), lambda i: (0, i, 0)),
            pl.BlockSpec((Hn, nb, 32), lambda i: (0, i, 0)),
        ],
        out_shape=[
            jax.ShapeDtypeStruct((Hn, N_PAD, 32), jnp.float32),
            jax.ShapeDtypeStruct((Hn, N_PAD, 32), jnp.float32),
        ],
    )(outp.reshape(HEADS, N_PAD, 32), bias.reshape(1, 128), Wl, Wr)


def _focal_body(op3_ref, focal_ref, o_ref, fv, r0, sem):
    wid = lax.axis_index("s") * NC + lax.axis_index("c")

    @pl.when(wid == 0)
    def _():
        pltpu.sync_copy(focal_ref, fv)
        pltpu.async_copy(op3_ref.at[fv], r0, sem).wait()
        pltpu.sync_copy(r0, o_ref)


def _focal_gather(outp3, focal_idx):
    f = pl.kernel(
        _focal_body,
        out_type=jax.ShapeDtypeStruct((64, 32), jnp.float32),
        mesh=_mesh(),
        compiler_params=_sc_params(),
        scratch_types=[
            pltpu.VMEM((64,), jnp.int32),
            pltpu.VMEM((64, 32), jnp.float32),
            pltpu.SemaphoreType.DMA,
        ],
    )
    return f(outp3, focal_idx.astype(jnp.int32))


def _fc_body(rows_ref, b3_ref, w_ref, bfc_ref, o_ref):
    comb = rows_ref[...] + b3_ref[...]
    o_ref[...] = jnp.dot(comb, w_ref[...],
                         preferred_element_type=jnp.float32) + bfc_ref[...]


def _fc(focal_idx, outp3, b3, Wfc_top, bfc):
    rows = _focal_gather(outp3, focal_idx)
    return pl.pallas_call(
        _fc_body,
        out_shape=jax.ShapeDtypeStruct((64, 60), jnp.float32),
    )(rows, b3.reshape(1, 32), Wfc_top, bfc.reshape(1, 60))


def _gat_layer(xl, xr, src, dst, att, H):
    xl_flat = xl.reshape(H * N_PAD, 32)
    xr_flat = xr.reshape(H * N_PAD, 32)
    ex, dpart = _edge_b(xl_flat, xr_flat, src, dst, att, H)
    den = _denom_combine(dpart, H)
    return _edge_d(xl_flat, src, dst, ex, den, H)


def kernel(x, edge_index, focal_idx, Wemb, bemb, W1l, W1r, att1, b1,
           W2l, W2r, att2, b2, W3l, W3r, att3, b3, Wfc, bfc):
    N = x.shape[0]
    loops = jnp.arange(N, dtype=jnp.int32)
    src = jnp.concatenate([edge_index[0].astype(jnp.int32), loops])
    dst = jnp.concatenate([edge_index[1].astype(jnp.int32), loops])
    pad = E_PAD - E_TOT
    src = jnp.concatenate([src, jnp.full((pad,), N, jnp.int32)])
    dst = jnp.concatenate([dst, jnp.full((pad,), N, jnp.int32)])

    x2 = x.reshape(N, 10)
    x2p = jnp.zeros((N_PAD, 10), jnp.float32).at[:N].set(x2)

    xl1, xr1 = _embed(x2p, Wemb, bemb, W1l, W1r)
    op1 = _gat_layer(xl1, xr1, src, dst, att1, HEADS)
    xl2, xr2 = _combine_e(op1, b1, W2l, W2r, HEADS)
    op2 = _gat_layer(xl2, xr2, src, dst, att2, HEADS)
    xl3, xr3 = _combine_e(op2, b2, W3l, W3r, 1)
    op3 = _gat_layer(xl3, xr3, src, dst, att3, 1)
    return _fc(focal_idx, op3, b3, Wfc[:HID], bfc)
